# Initial kernel scaffold; baseline (speedup 1.0000x reference)
import jax
import jax.numpy as jnp
from jax.experimental import pallas as pl


def kernel(t, P):
    raise NotImplementedError("write your pallas kernel here")


# speedup vs baseline: 4.1564x; 4.1564x over previous
import functools

import jax
import jax.numpy as jnp
from jax import lax
from jax.experimental import pallas as pl
from jax.experimental.pallas import tpu as pltpu
from jax.experimental.pallas import tpu_sc as plsc

_EMBED = 64
_NC = 2
_NS = 16
_NW = _NC * _NS
_CHUNK = 128


def _sc_gather(t3, P, n_chunks):
    total = _NW * n_chunks * _CHUNK
    per_w = n_chunks * _CHUNK
    mesh = plsc.VectorSubcoreMesh(core_axis_name="c", subcore_axis_name="s")

    @functools.partial(
        pl.kernel,
        mesh=mesh,
        out_type=jax.ShapeDtypeStruct((total, _EMBED), jnp.float32),
        scratch_types=[
            pltpu.VMEM((n_chunks, _CHUNK), jnp.int32),
            pltpu.VMEM((_CHUNK, _EMBED), jnp.float32),
            pltpu.SemaphoreType.DMA,
        ],
        compiler_params=pltpu.CompilerParams(use_tc_tiling_on_sc=False),
    )
    def k(t_hbm, P_hbm, out_hbm, idx_v, rows_v, sem):
        wid = lax.axis_index("s") * _NC + lax.axis_index("c")
        base = wid * per_w
        pltpu.sync_copy(t_hbm.at[wid], idx_v)

        def body(g, carry):
            pltpu.async_copy(P_hbm.at[idx_v.at[g]], rows_v, sem).wait()
            pltpu.sync_copy(rows_v, out_hbm.at[pl.ds(base + g * _CHUNK, _CHUNK)])
            return carry

        lax.fori_loop(0, n_chunks, body, 0)

    return k(t3, P)


def kernel(t, P):
    B, H = t.shape
    flat = B * H
    n_chunks = flat // (_NW * _CHUNK)
    t3 = t.reshape(_NW, n_chunks, _CHUNK)
    out = _sc_gather(t3, P, n_chunks)
    return out.reshape(B, H, _EMBED)


# speedup vs baseline: 4.8070x; 1.1566x over previous
import functools

import jax
import jax.numpy as jnp
from jax import lax
from jax.experimental import pallas as pl
from jax.experimental.pallas import tpu as pltpu
from jax.experimental.pallas import tpu_sc as plsc

_EMBED = 64
_NC = 2
_NS = 16
_NW = _NC * _NS
_CHUNK = 512


def _sc_gather(t3, P, n_chunks):
    total = _NW * n_chunks * _CHUNK
    per_w = n_chunks * _CHUNK
    mesh = plsc.VectorSubcoreMesh(core_axis_name="c", subcore_axis_name="s")

    @functools.partial(
        pl.kernel,
        mesh=mesh,
        out_type=jax.ShapeDtypeStruct((total, _EMBED), jnp.float32),
        scratch_types=[
            pltpu.VMEM((n_chunks, _CHUNK), jnp.int32),
            pltpu.VMEM((_CHUNK, _EMBED), jnp.float32),
            pltpu.SemaphoreType.DMA,
        ],
        compiler_params=pltpu.CompilerParams(use_tc_tiling_on_sc=False),
    )
    def k(t_hbm, P_hbm, out_hbm, idx_v, rows_v, sem):
        wid = lax.axis_index("s") * _NC + lax.axis_index("c")
        base = wid * per_w
        pltpu.sync_copy(t_hbm.at[wid], idx_v)

        def body(g, carry):
            pltpu.async_copy(P_hbm.at[idx_v.at[g]], rows_v, sem).wait()
            pltpu.sync_copy(rows_v, out_hbm.at[pl.ds(base + g * _CHUNK, _CHUNK)])
            return carry

        lax.fori_loop(0, n_chunks, body, 0)

    return k(t3, P)


def kernel(t, P):
    B, H = t.shape
    flat = B * H
    n_chunks = flat // (_NW * _CHUNK)
    t3 = t.reshape(_NW, n_chunks, _CHUNK)
    out = _sc_gather(t3, P, n_chunks)
    return out.reshape(B, H, _EMBED)


# speedup vs baseline: 4.9436x; 1.0284x over previous
import functools

import jax
import jax.numpy as jnp
from jax import lax
from jax.experimental import pallas as pl
from jax.experimental.pallas import tpu as pltpu
from jax.experimental.pallas import tpu_sc as plsc

_EMBED = 64
_NC = 2
_NS = 16
_NW = _NC * _NS
_CHUNK = 512
_NSLOT = 3


def _sc_gather(t3, P, n_chunks):
    total = _NW * n_chunks * _CHUNK
    per_w = n_chunks * _CHUNK
    mesh = plsc.VectorSubcoreMesh(core_axis_name="c", subcore_axis_name="s")
    n_pad = ((n_chunks + _NSLOT - 1) // _NSLOT) * _NSLOT

    @functools.partial(
        pl.kernel,
        mesh=mesh,
        out_type=jax.ShapeDtypeStruct((total, _EMBED), jnp.float32),
        scratch_types=[
            pltpu.VMEM((n_chunks, _CHUNK), jnp.int32),
            pltpu.VMEM((_NSLOT, _CHUNK, _EMBED), jnp.float32),
            [pltpu.SemaphoreType.DMA] * _NSLOT,
            [pltpu.SemaphoreType.DMA] * _NSLOT,
        ],
        compiler_params=pltpu.CompilerParams(use_tc_tiling_on_sc=False),
    )
    def k(t_hbm, P_hbm, out_hbm, idx_v, rows_v, gsems, wsems):
        wid = lax.axis_index("s") * _NC + lax.axis_index("c")
        base = wid * per_w
        pltpu.sync_copy(t_hbm.at[wid], idx_v)

        def gather_start(c, b):
            pltpu.async_copy(P_hbm.at[idx_v.at[c]], rows_v.at[b], gsems[b])

        def gather_wait(b):
            pltpu.make_async_copy(
                P_hbm.at[pl.ds(0, _CHUNK)], rows_v.at[b], gsems[b]
            ).wait()

        def write_start(c, b):
            pltpu.async_copy(
                rows_v.at[b], out_hbm.at[pl.ds(base + c * _CHUNK, _CHUNK)], wsems[b]
            )

        def write_wait(b):
            pltpu.make_async_copy(
                P_hbm.at[pl.ds(0, _CHUNK)], rows_v.at[b], wsems[b]
            ).wait()

        for b in range(_NSLOT):
            gather_start(b, b)

        @pl.loop(0, n_pad, step=_NSLOT)
        def _body(c0):
            for b in range(_NSLOT):
                c = c0 + b

                @pl.when(c < n_chunks)
                def _():
                    gather_wait(b)
                    write_start(c, b)

            for b in range(_NSLOT):
                c = c0 + b + _NSLOT

                @pl.when(c < n_chunks)
                def _():
                    write_wait(b)
                    gather_start(c, b)

        for b in range(_NSLOT):
            write_wait(b)

    return k(t3, P)


def kernel(t, P):
    B, H = t.shape
    flat = B * H
    n_chunks = flat // (_NW * _CHUNK)
    t3 = t.reshape(_NW, n_chunks, _CHUNK)
    out = _sc_gather(t3, P, n_chunks)
    return out.reshape(B, H, _EMBED)


# speedup vs baseline: 5.6178x; 1.1364x over previous
import functools

import jax
import jax.numpy as jnp
from jax import lax
from jax.experimental import pallas as pl
from jax.experimental.pallas import tpu as pltpu
from jax.experimental.pallas import tpu_sc as plsc

_EMBED = 64
_NC = 2
_NS = 16
_NW = _NC * _NS
_CHUNK = 256
_NSLOT = 4


def _sc_gather(t3, P, n_chunks):
    total = _NW * n_chunks * _CHUNK
    per_w = n_chunks * _CHUNK
    mesh = plsc.VectorSubcoreMesh(core_axis_name="c", subcore_axis_name="s")
    n_pad = ((n_chunks + _NSLOT - 1) // _NSLOT) * _NSLOT

    @functools.partial(
        pl.kernel,
        mesh=mesh,
        out_type=jax.ShapeDtypeStruct((total, _EMBED), jnp.float32),
        scratch_types=[
            pltpu.VMEM((n_chunks, _CHUNK), jnp.int32),
            pltpu.VMEM((_NSLOT, _CHUNK, _EMBED), jnp.float32),
            pltpu.VMEM_SHARED((8192, _EMBED), jnp.float32),
            [pltpu.SemaphoreType.DMA] * _NSLOT,
            [pltpu.SemaphoreType.DMA] * _NSLOT,
        ],
        compiler_params=pltpu.CompilerParams(use_tc_tiling_on_sc=False),
    )
    def k(t_hbm, P_hbm, out_hbm, idx_v, rows_v, table_sh, gsems, wsems):
        sid = lax.axis_index("s")
        wid = sid * _NC + lax.axis_index("c")
        base = wid * per_w
        shard = 8192 // _NS
        pltpu.sync_copy(
            P_hbm.at[pl.ds(sid * shard, shard)],
            table_sh.at[pl.ds(sid * shard, shard)],
        )
        pltpu.sync_copy(t_hbm.at[wid], idx_v)
        plsc.subcore_barrier()

        def gather_start(c, b):
            pltpu.async_copy(table_sh.at[idx_v.at[c]], rows_v.at[b], gsems[b])

        def gather_wait(b):
            pltpu.make_async_copy(
                P_hbm.at[pl.ds(0, _CHUNK)], rows_v.at[b], gsems[b]
            ).wait()

        def write_start(c, b):
            pltpu.async_copy(
                rows_v.at[b], out_hbm.at[pl.ds(base + c * _CHUNK, _CHUNK)], wsems[b]
            )

        def write_wait(b):
            pltpu.make_async_copy(
                P_hbm.at[pl.ds(0, _CHUNK)], rows_v.at[b], wsems[b]
            ).wait()

        for b in range(_NSLOT):
            gather_start(b, b)

        @pl.loop(0, n_pad, step=_NSLOT)
        def _body(c0):
            for b in range(_NSLOT):
                c = c0 + b

                @pl.when(c < n_chunks)
                def _():
                    gather_wait(b)
                    write_start(c, b)

            for b in range(_NSLOT):
                c = c0 + b + _NSLOT

                @pl.when(c < n_chunks)
                def _():
                    write_wait(b)
                    gather_start(c, b)

        for b in range(_NSLOT):
            write_wait(b)

    return k(t3, P)


def kernel(t, P):
    B, H = t.shape
    flat = B * H
    n_chunks = flat // (_NW * _CHUNK)
    t3 = t.reshape(_NW, n_chunks, _CHUNK)
    out = _sc_gather(t3, P, n_chunks)
    return out.reshape(B, H, _EMBED)
